# Initial kernel scaffold; baseline (speedup 1.0000x reference)
import jax
import jax.numpy as jnp
from jax.experimental import pallas as pl


def kernel(x, edge_index, edge_attr, w11, b11, w21, b21, root1, bias1, w12, b12, w22, b22, root2, bias2, fc_w, fc_b):
    raise NotImplementedError("write your pallas kernel here")


# speedup vs baseline: 1.1627x; 1.1627x over previous
import functools

import jax
import jax.numpy as jnp
from jax import lax
from jax.experimental import pallas as pl
from jax.experimental.pallas import tpu as pltpu

N, E, DIN, DE, H = 10000, 160000, 128, 10, 8
EB = 1600
NB = 1000


def _edge1_body(ea_ref, xj_ref, w11_ref, b11_ref, w2m_ref, b2r_ref, s_ref,
                out_ref):
    ea = ea_ref[...]
    xj = xj_ref[...]
    h = jnp.maximum(ea @ w11_ref[...] + b11_ref[...], 0.0)
    u = xj @ w2m_ref[...]
    z = u * jnp.tile(h, (1, H))
    m = z @ s_ref[...] + xj @ b2r_ref[...]
    nb = ea.shape[0]
    out_ref[...] = jnp.concatenate(
        [m, jnp.ones((nb, 1), jnp.float32),
         jnp.zeros((nb, 16 - H - 1), jnp.float32)], axis=1)


def _edge2_body(ea_ref, xj_ref, w12_ref, b12_ref, w2m_ref, b2r_ref, s_ref,
                out_ref):
    ea = ea_ref[...]
    xj = xj_ref[...][:, :H]
    h = jnp.maximum(ea @ w12_ref[...] + b12_ref[...], 0.0)
    u = xj @ w2m_ref[...]
    z = u * jnp.tile(h, (1, H // 2))
    m = z @ s_ref[...] + xj @ b2r_ref[...]
    out_ref[...] = jnp.concatenate(
        [m, jnp.zeros((ea.shape[0], 16 - H // 2), jnp.float32)], axis=1)


def _node1_body(s_ref, x_ref, root_ref, bias_ref, root2_ref, bias2_ref,
                x1p_ref, r2_ref):
    s = s_ref[...]
    cnt = jnp.maximum(s[:, H:H + 1], 1.0)
    aggr = s[:, :H] / cnt
    x1 = jnp.maximum(aggr + x_ref[...] @ root_ref[...] + bias_ref[...], 0.0)
    x1p_ref[...] = jnp.concatenate(
        [x1, jnp.zeros((s.shape[0], 16 - H), jnp.float32)], axis=1)
    r2_ref[...] = x1 @ root2_ref[...] + bias2_ref[...]


def _node2_body(s_ref, s1_ref, r2_ref, fca_ref, fcb_ref, ya_ref, yb_ref):
    s = s_ref[...]
    cnt = jnp.maximum(s1_ref[...][:, H:H + 1], 1.0)
    aggr = s[:, :H // 2] / cnt
    x2 = jnp.maximum(aggr + r2_ref[...], 0.0)
    pad = jnp.zeros((s.shape[0], 15), jnp.float32)
    ya_ref[...] = jnp.concatenate([x2 @ fca_ref[...], pad], axis=1)
    yb_ref[...] = jnp.concatenate([x2 @ fcb_ref[...], pad], axis=1)


def _final_body(g1_ref, g2_ref, fcb_ref, out_ref):
    z = g1_ref[...][:, 0:1] + g2_ref[...][:, 0:1] + fcb_ref[...]
    out_ref[...] = jax.nn.sigmoid(z)


def _full(i):
    return pl.BlockSpec((None,), lambda *_: (0,))


def kernel(x, edge_index, edge_attr, w11, b11, w21, b21, root1, bias1,
           w12, b12, w22, b22, root2, bias2, fc_w, fc_b):
    f32 = jnp.float32
    src = edge_index[0]
    dst = edge_index[1]

    w2m1 = w21.reshape(H * H, DIN, H).transpose(1, 2, 0).reshape(DIN, 512)
    b2r1 = b21.reshape(DIN, H)
    w2m2 = w22.reshape(H * H, H, H // 2).transpose(1, 2, 0).reshape(H, 256)
    b2r2 = b22.reshape(H, H // 2)
    s1m = (jnp.arange(512)[:, None] // 64 == jnp.arange(H)[None, :]).astype(f32)
    s2m = (jnp.arange(256)[:, None] // 64
           == jnp.arange(H // 2)[None, :]).astype(f32)
    b11r = b11.reshape(1, H * H)
    b12r = b12.reshape(1, H * H)
    bias1r = bias1.reshape(1, H)
    bias2r = bias2.reshape(1, H // 2)
    fca = fc_w[:H // 2]
    fcb = fc_w[H // 2:]

    xj = x[src]

    wspec = pl.BlockSpec(lambda i: (0, 0))

    def edge_call(body, xj_arr, *weights):
        n_w = len(weights)
        return pl.pallas_call(
            body,
            grid=(E // EB,),
            in_specs=[
                pl.BlockSpec((EB, DE), lambda i: (i, 0)),
                pl.BlockSpec((EB, xj_arr.shape[1]), lambda i: (i, 0)),
            ] + [pl.BlockSpec(w.shape, lambda i: (0,) * w.ndim)
                 for w in weights],
            out_specs=pl.BlockSpec((EB, 16), lambda i: (i, 0)),
            out_shape=jax.ShapeDtypeStruct((E, 16), f32),
        )(edge_attr, xj_arr, *weights)

    m1 = edge_call(_edge1_body, xj, w11, b11r, w2m1, b2r1, s1m)

    s1 = jax.ops.segment_sum(m1, dst, num_segments=N)

    x1p, r2 = pl.pallas_call(
        _node1_body,
        grid=(N // NB,),
        in_specs=[
            pl.BlockSpec((NB, 16), lambda i: (i, 0)),
            pl.BlockSpec((NB, DIN), lambda i: (i, 0)),
            pl.BlockSpec(root1.shape, lambda i: (0, 0)),
            pl.BlockSpec(bias1r.shape, lambda i: (0, 0)),
            pl.BlockSpec(root2.shape, lambda i: (0, 0)),
            pl.BlockSpec(bias2r.shape, lambda i: (0, 0)),
        ],
        out_specs=[
            pl.BlockSpec((NB, 16), lambda i: (i, 0)),
            pl.BlockSpec((NB, H // 2), lambda i: (i, 0)),
        ],
        out_shape=[
            jax.ShapeDtypeStruct((N, 16), f32),
            jax.ShapeDtypeStruct((N, H // 2), f32),
        ],
    )(s1, x, root1, bias1r, root2, bias2r)

    xj1 = x1p[src]

    m2 = edge_call(_edge2_body, xj1, w12, b12r, w2m2, b2r2, s2m)

    s2 = jax.ops.segment_sum(m2, dst, num_segments=N)

    ya, yb = pl.pallas_call(
        _node2_body,
        grid=(N // NB,),
        in_specs=[
            pl.BlockSpec((NB, 16), lambda i: (i, 0)),
            pl.BlockSpec((NB, 16), lambda i: (i, 0)),
            pl.BlockSpec((NB, H // 2), lambda i: (i, 0)),
            pl.BlockSpec(fca.shape, lambda i: (0, 0)),
            pl.BlockSpec(fcb.shape, lambda i: (0, 0)),
        ],
        out_specs=[
            pl.BlockSpec((NB, 16), lambda i: (i, 0)),
            pl.BlockSpec((NB, 16), lambda i: (i, 0)),
        ],
        out_shape=[
            jax.ShapeDtypeStruct((N, 16), f32),
            jax.ShapeDtypeStruct((N, 16), f32),
        ],
    )(s2, s1, r2, fca, fcb)

    g1 = ya[src]
    g2 = yb[dst]

    out = pl.pallas_call(
        _final_body,
        grid=(E // EB,),
        in_specs=[
            pl.BlockSpec((EB, 16), lambda i: (i, 0)),
            pl.BlockSpec((EB, 16), lambda i: (i, 0)),
            pl.BlockSpec((1, 1), lambda i: (0, 0)),
        ],
        out_specs=pl.BlockSpec((EB, 1), lambda i: (i, 0)),
        out_shape=jax.ShapeDtypeStruct((E, 1), f32),
    )(g1, g2, fc_b.reshape(1, 1))
    return out


# speedup vs baseline: 2.7038x; 2.3254x over previous
import functools

import jax
import jax.numpy as jnp
from jax import lax
from jax.experimental import pallas as pl
from jax.experimental.pallas import tpu as pltpu
from jax.experimental.pallas import tpu_sc as plsc

N, E, DIN, DE, H = 10000, 160000, 128, 10, 8
EB = 1600
NB = 1000
NC, NS = 2, 16
NW = NC * NS
EPAD = 163840
PERW = EPAD // NW
NPAD = 10112
STRIPE = NPAD // NS
W128 = 128


def _edge1_body(ea_ref, xj_ref, w11_ref, b11_ref, w2m_ref, b2r_ref, s_ref,
                out_ref):
    ea = ea_ref[...]
    xj = xj_ref[...]
    h = jnp.maximum(ea @ w11_ref[...] + b11_ref[...], 0.0)
    u = xj @ w2m_ref[...]
    z = u * jnp.tile(h, (1, H))
    m = z @ s_ref[...] + xj @ b2r_ref[...]
    nb = ea.shape[0]
    out_ref[...] = jnp.concatenate(
        [m, jnp.ones((nb, 1), jnp.float32),
         jnp.zeros((nb, W128 - H - 1), jnp.float32)], axis=1)


def _edge2_body(ea_ref, xj_ref, w12_ref, b12_ref, w2m_ref, b2r_ref, s_ref,
                out_ref):
    ea = ea_ref[...]
    xj = xj_ref[...][:, :H]
    h = jnp.maximum(ea @ w12_ref[...] + b12_ref[...], 0.0)
    u = xj @ w2m_ref[...]
    z = u * jnp.tile(h, (1, H // 2))
    m = z @ s_ref[...] + xj @ b2r_ref[...]
    out_ref[...] = jnp.concatenate(
        [m, jnp.zeros((ea.shape[0], W128 - H // 2), jnp.float32)], axis=1)


def _node1_body(sa_ref, sb_ref, x_ref, root_ref, bias_ref, root2_ref,
                bias2_ref, x1p_ref, r2_ref):
    sa = sa_ref[...]
    sb = sb_ref[...]
    cnt = jnp.maximum(sa[:, H:H + 1] + sb[:, H:H + 1], 1.0)
    aggr = (sa[:, :H] + sb[:, :H]) / cnt
    x1 = jnp.maximum(aggr + x_ref[...] @ root_ref[...] + bias_ref[...], 0.0)
    x1p_ref[...] = jnp.concatenate(
        [x1, jnp.zeros((sa.shape[0], W128 - H), jnp.float32)], axis=1)
    r2_ref[...] = x1 @ root2_ref[...] + bias2_ref[...]


def _node2_body(sa_ref, sb_ref, ca_ref, cb_ref, r2_ref, fca_ref, fcb_ref,
                fcbias_ref, ya_ref, yb_ref):
    sa = sa_ref[...]
    sb = sb_ref[...]
    cnt = jnp.maximum(ca_ref[...][:, H:H + 1] + cb_ref[...][:, H:H + 1], 1.0)
    aggr = (sa[:, :H // 2] + sb[:, :H // 2]) / cnt
    x2 = jnp.maximum(aggr + r2_ref[...], 0.0)
    pad = jnp.zeros((sa.shape[0], 15), jnp.float32)
    ya_ref[...] = jnp.concatenate([x2 @ fca_ref[...], pad], axis=1)
    yb_ref[...] = jnp.concatenate(
        [x2 @ fcb_ref[...] + fcbias_ref[...], pad], axis=1)


def _sc_mesh():
    return plsc.VectorSubcoreMesh(core_axis_name="c", subcore_axis_name="s")


def _sc_gather(table, idx2d, chunk):
    sub = chunk // 128
    nhalf = 1024 // chunk

    @functools.partial(
        pl.kernel,
        out_type=jax.ShapeDtypeStruct((EPAD, W128), jnp.float32),
        mesh=_sc_mesh(),
        scratch_types=[
            pltpu.VMEM((8, 128), jnp.int32),
            pltpu.VMEM((chunk, W128), jnp.float32),
            pltpu.SemaphoreType.DMA,
        ],
    )
    def k(table_h, idx_h, out_h, idx_v, rows_v, sem):
        wid = lax.axis_index("c") * NS + lax.axis_index("s")
        for t in range(PERW // 1024):
            rowb = wid * (PERW // 128) + t * 8
            pltpu.sync_copy(idx_h.at[pl.ds(rowb, 8)], idx_v)
            for hh in range(nhalf):
                base = wid * PERW + t * 1024 + hh * chunk
                descs = [
                    pltpu.async_copy(table_h.at[idx_v.at[hh * sub + j]],
                                     rows_v.at[pl.ds(j * 128, 128)], sem)
                    for j in range(sub)
                ]
                for d in descs:
                    d.wait()
                pltpu.sync_copy(rows_v, out_h.at[pl.ds(base, chunk)])

    return k(table, idx2d)


def _sc_scatter_sum(m_pad, dst2d, zer):
    @functools.partial(
        pl.kernel,
        out_type=jax.ShapeDtypeStruct((NC, NPAD, W128), jnp.float32),
        mesh=_sc_mesh(),
        scratch_types=[
            pltpu.VMEM((8, 128), jnp.int32),
            pltpu.VMEM((256, W128), jnp.float32),
            pltpu.VMEM_SHARED((NPAD, W128), jnp.float32),
            pltpu.SemaphoreType.DMA,
        ],
    )
    def k(m_h, dst_h, zer_h, out_h, idx_v, m_v, acc_sh, sem):
        c = lax.axis_index("c")
        s = lax.axis_index("s")
        wid = c * NS + s
        pltpu.sync_copy(zer_h.at[pl.ds(s * STRIPE, STRIPE)],
                        acc_sh.at[pl.ds(s * STRIPE, STRIPE)])
        plsc.subcore_barrier()
        for t in range(PERW // 1024):
            rowb = wid * (PERW // 128) + t * 8
            pltpu.sync_copy(dst_h.at[pl.ds(rowb, 8)], idx_v)
            for hh in range(4):
                base = wid * PERW + t * 1024 + hh * 256
                pltpu.sync_copy(m_h.at[pl.ds(base, 256)], m_v)
                for j in range(2):
                    pltpu.sync_copy(m_v.at[pl.ds(j * 128, 128)],
                                    acc_sh.at[idx_v.at[hh * 2 + j]], add=True)
        plsc.subcore_barrier()
        pltpu.sync_copy(acc_sh.at[pl.ds(s * STRIPE, STRIPE)],
                        out_h.at[c, pl.ds(s * STRIPE, STRIPE)])

    return k(m_pad, dst2d, zer)


def _sc_final(ya1, yb1, src2d, dst2d):
    @functools.partial(
        pl.kernel,
        out_type=jax.ShapeDtypeStruct((EPAD,), jnp.float32),
        mesh=_sc_mesh(),
        scratch_types=[
            pltpu.VMEM((8, 128), jnp.int32),
            pltpu.VMEM((8, 128), jnp.int32),
            pltpu.VMEM((1024,), jnp.float32),
            pltpu.VMEM((1024,), jnp.float32),
            pltpu.SemaphoreType.DMA,
        ],
        compiler_params=pltpu.CompilerParams(use_tc_tiling_on_sc=False),
    )
    def k(ya_h, yb_h, si_h, di_h, out_h, ia_v, ib_v, va_v, vb_v, sem):
        wid = lax.axis_index("c") * NS + lax.axis_index("s")
        for t in range(PERW // 1024):
            base = wid * PERW + t * 1024
            rowb = wid * (PERW // 128) + t * 8
            pltpu.sync_copy(si_h.at[pl.ds(rowb, 8)], ia_v)
            pltpu.sync_copy(di_h.at[pl.ds(rowb, 8)], ib_v)
            descs = []
            for j in range(8):
                descs.append(
                    pltpu.async_copy(ya_h.at[ia_v.at[j]],
                                     va_v.at[pl.ds(j * 128, 128)], sem))
                descs.append(
                    pltpu.async_copy(yb_h.at[ib_v.at[j]],
                                     vb_v.at[pl.ds(j * 128, 128)], sem))
            for d in descs:
                d.wait()

            def body(i, carry):
                off = pl.multiple_of(i * 16, 16)
                z = va_v[pl.ds(off, 16)] + vb_v[pl.ds(off, 16)]
                va_v[pl.ds(off, 16)] = 1.0 / (1.0 + jnp.exp(-z))
                return carry

            lax.fori_loop(0, 1024 // 16, body, 0)
            pltpu.sync_copy(va_v, out_h.at[pl.ds(base, 1024)])

    return k(ya1, yb1, src2d, dst2d)


def kernel(x, edge_index, edge_attr, w11, b11, w21, b21, root1, bias1,
           w12, b12, w22, b22, root2, bias2, fc_w, fc_b):
    f32 = jnp.float32
    i32 = jnp.int32
    src = edge_index[0].astype(i32)
    dst = edge_index[1].astype(i32)

    w2m1 = w21.reshape(H * H, DIN, H).transpose(1, 2, 0).reshape(DIN, 512)
    b2r1 = b21.reshape(DIN, H)
    w2m2 = w22.reshape(H * H, H, H // 2).transpose(1, 2, 0).reshape(H, 256)
    b2r2 = b22.reshape(H, H // 2)
    s1m = (jnp.arange(512)[:, None] // 64 == jnp.arange(H)[None, :]).astype(f32)
    s2m = (jnp.arange(256)[:, None] // 64
           == jnp.arange(H // 2)[None, :]).astype(f32)
    b11r = b11.reshape(1, H * H)
    b12r = b12.reshape(1, H * H)
    bias1r = bias1.reshape(1, H)
    bias2r = bias2.reshape(1, H // 2)
    fca = fc_w[:H // 2]
    fcb = fc_w[H // 2:]
    fcbias = fc_b.reshape(1, 1)

    pad = EPAD - E
    src2d = jnp.concatenate([src, jnp.zeros((pad,), i32)]).reshape(
        EPAD // 128, 128)
    dst2d = jnp.concatenate([dst, jnp.full((pad,), N, i32)]).reshape(
        EPAD // 128, 128)
    zer = jnp.zeros((NPAD, W128), f32)

    xj = _sc_gather(x, src2d, 512)

    def edge_call(body, xj_arr, *weights):
        return pl.pallas_call(
            body,
            grid=(E // EB,),
            in_specs=[
                pl.BlockSpec((EB, DE), lambda i: (i, 0)),
                pl.BlockSpec((EB, W128), lambda i: (i, 0)),
            ] + [pl.BlockSpec(w.shape, lambda i: (0,) * w.ndim)
                 for w in weights],
            out_specs=pl.BlockSpec((EB, W128), lambda i: (i, 0)),
            out_shape=jax.ShapeDtypeStruct((EPAD, W128), f32),
        )(edge_attr, xj_arr, *weights)

    m1 = edge_call(_edge1_body, xj, w11, b11r, w2m1, b2r1, s1m)

    s1 = _sc_scatter_sum(m1, dst2d, zer)

    half_a = pl.BlockSpec((None, NB, W128), lambda i: (0, i, 0))
    half_b = pl.BlockSpec((None, NB, W128), lambda i: (1, i, 0))

    x1p, r2 = pl.pallas_call(
        _node1_body,
        grid=(N // NB,),
        in_specs=[
            half_a, half_b,
            pl.BlockSpec((NB, DIN), lambda i: (i, 0)),
            pl.BlockSpec(root1.shape, lambda i: (0, 0)),
            pl.BlockSpec(bias1r.shape, lambda i: (0, 0)),
            pl.BlockSpec(root2.shape, lambda i: (0, 0)),
            pl.BlockSpec(bias2r.shape, lambda i: (0, 0)),
        ],
        out_specs=[
            pl.BlockSpec((NB, W128), lambda i: (i, 0)),
            pl.BlockSpec((NB, H // 2), lambda i: (i, 0)),
        ],
        out_shape=[
            jax.ShapeDtypeStruct((N, W128), f32),
            jax.ShapeDtypeStruct((N, H // 2), f32),
        ],
    )(s1, s1, x, root1, bias1r, root2, bias2r)

    xj1 = _sc_gather(x1p, src2d, 512)

    m2 = edge_call(_edge2_body, xj1, w12, b12r, w2m2, b2r2, s2m)

    s2 = _sc_scatter_sum(m2, dst2d, zer)

    ya2d, yb2d = pl.pallas_call(
        _node2_body,
        grid=(N // NB,),
        in_specs=[
            half_a, half_b, half_a, half_b,
            pl.BlockSpec((NB, H // 2), lambda i: (i, 0)),
            pl.BlockSpec(fca.shape, lambda i: (0, 0)),
            pl.BlockSpec(fcb.shape, lambda i: (0, 0)),
            pl.BlockSpec(fcbias.shape, lambda i: (0, 0)),
        ],
        out_specs=[
            pl.BlockSpec((NB, 16), lambda i: (i, 0)),
            pl.BlockSpec((NB, 16), lambda i: (i, 0)),
        ],
        out_shape=[
            jax.ShapeDtypeStruct((NPAD, 16), f32),
            jax.ShapeDtypeStruct((NPAD, 16), f32),
        ],
    )(s2, s2, s1, s1, r2, fca, fcb, fcbias)

    out = _sc_final(ya2d[:, 0], yb2d[:, 0], src2d, dst2d)
    return out[:E].reshape(E, 1)


# speedup vs baseline: 3.0718x; 1.1361x over previous
import functools

import jax
import jax.numpy as jnp
from jax import lax
from jax.experimental import pallas as pl
from jax.experimental.pallas import tpu as pltpu
from jax.experimental.pallas import tpu_sc as plsc

N, E, DIN, DE, H = 10000, 160000, 128, 10, 8
EB = 1600
NB = 1000
NC, NS = 2, 16
NW = NC * NS
EPAD = 163840
PERW = EPAD // NW
NPAD = 10112
STRIPE = NPAD // NS
W128 = 128
W16 = 16


def _edge1_body(ea_ref, xj_ref, w11_ref, b11_ref, w2m_ref, b2r_ref, s_ref,
                out_ref):
    ea = ea_ref[...]
    xj = xj_ref[...]
    h = jnp.maximum(ea @ w11_ref[...] + b11_ref[...], 0.0)
    u = xj @ w2m_ref[...]
    z = u * jnp.tile(h, (1, H))
    m = z @ s_ref[...] + xj @ b2r_ref[...]
    nb = ea.shape[0]
    out_ref[...] = jnp.concatenate(
        [m, jnp.ones((nb, 1), jnp.float32),
         jnp.zeros((nb, W16 - H - 1), jnp.float32)], axis=1)


def _edge2_body(ea_ref, xj_ref, w12_ref, b12_ref, w2m_ref, b2r_ref, s_ref,
                out_ref):
    ea = ea_ref[...]
    xj = xj_ref[...][:, :H]
    h = jnp.maximum(ea @ w12_ref[...] + b12_ref[...], 0.0)
    u = xj @ w2m_ref[...]
    z = u * jnp.tile(h, (1, H // 2))
    m = z @ s_ref[...] + xj @ b2r_ref[...]
    out_ref[...] = jnp.concatenate(
        [m, jnp.zeros((ea.shape[0], W16 - H // 2), jnp.float32)], axis=1)


def _node1_body(sa_ref, sb_ref, x_ref, root_ref, bias_ref, root2_ref,
                bias2_ref, x1p_ref, r2_ref):
    sa = sa_ref[...]
    sb = sb_ref[...]
    cnt = jnp.maximum(sa[:, H:H + 1] + sb[:, H:H + 1], 1.0)
    aggr = (sa[:, :H] + sb[:, :H]) / cnt
    x1 = jnp.maximum(aggr + x_ref[...] @ root_ref[...] + bias_ref[...], 0.0)
    x1p_ref[...] = jnp.concatenate(
        [x1, jnp.zeros((sa.shape[0], W16 - H), jnp.float32)], axis=1)
    r2_ref[...] = x1 @ root2_ref[...] + bias2_ref[...]


def _node2_body(sa_ref, sb_ref, ca_ref, cb_ref, r2_ref, fca_ref, fcb_ref,
                fcbias_ref, ya_ref, yb_ref):
    sa = sa_ref[...]
    sb = sb_ref[...]
    cnt = jnp.maximum(ca_ref[...][:, H:H + 1] + cb_ref[...][:, H:H + 1], 1.0)
    aggr = (sa[:, :H // 2] + sb[:, :H // 2]) / cnt
    x2 = jnp.maximum(aggr + r2_ref[...], 0.0)
    pad = jnp.zeros((sa.shape[0], 15), jnp.float32)
    ya_ref[...] = jnp.concatenate([x2 @ fca_ref[...], pad], axis=1)
    yb_ref[...] = jnp.concatenate(
        [x2 @ fcb_ref[...] + fcbias_ref[...], pad], axis=1)


def _sc_mesh():
    return plsc.VectorSubcoreMesh(core_axis_name="c", subcore_axis_name="s")


def _sc_gather(table, idx2d, chunk, width):
    sub = chunk // 128
    nhalf = 1024 // chunk
    params = {} if width == 128 else {
        "compiler_params": pltpu.CompilerParams(use_tc_tiling_on_sc=False)}

    @functools.partial(
        pl.kernel,
        out_type=jax.ShapeDtypeStruct((EPAD, width), jnp.float32),
        mesh=_sc_mesh(),
        scratch_types=[
            pltpu.VMEM((8, 128), jnp.int32),
            pltpu.VMEM((chunk, width), jnp.float32),
            pltpu.SemaphoreType.DMA,
        ],
        **params,
    )
    def k(table_h, idx_h, out_h, idx_v, rows_v, sem):
        wid = lax.axis_index("c") * NS + lax.axis_index("s")
        for t in range(PERW // 1024):
            rowb = wid * (PERW // 128) + t * 8
            pltpu.sync_copy(idx_h.at[pl.ds(rowb, 8)], idx_v)
            for hh in range(nhalf):
                base = wid * PERW + t * 1024 + hh * chunk
                descs = [
                    pltpu.async_copy(table_h.at[idx_v.at[hh * sub + j]],
                                     rows_v.at[pl.ds(j * 128, 128)], sem)
                    for j in range(sub)
                ]
                for d in descs:
                    d.wait()
                pltpu.sync_copy(rows_v, out_h.at[pl.ds(base, chunk)])

    return k(table, idx2d)


def _sc_scatter_sum(m_pad, dst2d, zer):
    @functools.partial(
        pl.kernel,
        out_type=jax.ShapeDtypeStruct((NC, NPAD, W16), jnp.float32),
        mesh=_sc_mesh(),
        scratch_types=[
            pltpu.VMEM((8, 128), jnp.int32),
            pltpu.VMEM((256, W16), jnp.float32),
            pltpu.VMEM_SHARED((NPAD, W16), jnp.float32),
            pltpu.SemaphoreType.DMA,
        ],
        compiler_params=pltpu.CompilerParams(use_tc_tiling_on_sc=False),
    )
    def k(m_h, dst_h, zer_h, out_h, idx_v, m_v, acc_sh, sem):
        c = lax.axis_index("c")
        s = lax.axis_index("s")
        wid = c * NS + s
        pltpu.sync_copy(zer_h.at[pl.ds(s * STRIPE, STRIPE)],
                        acc_sh.at[pl.ds(s * STRIPE, STRIPE)])
        plsc.subcore_barrier()
        for t in range(PERW // 1024):
            rowb = wid * (PERW // 128) + t * 8
            pltpu.sync_copy(dst_h.at[pl.ds(rowb, 8)], idx_v)
            for hh in range(4):
                base = wid * PERW + t * 1024 + hh * 256
                pltpu.sync_copy(m_h.at[pl.ds(base, 256)], m_v)
                for j in range(2):
                    pltpu.sync_copy(m_v.at[pl.ds(j * 128, 128)],
                                    acc_sh.at[idx_v.at[hh * 2 + j]], add=True)
        plsc.subcore_barrier()
        pltpu.sync_copy(acc_sh.at[pl.ds(s * STRIPE, STRIPE)],
                        out_h.at[c, pl.ds(s * STRIPE, STRIPE)])

    return k(m_pad, dst2d, zer)


def _sc_final(ya1, yb1, src2d, dst2d):
    @functools.partial(
        pl.kernel,
        out_type=jax.ShapeDtypeStruct((EPAD,), jnp.float32),
        mesh=_sc_mesh(),
        scratch_types=[
            pltpu.VMEM((8, 128), jnp.int32),
            pltpu.VMEM((8, 128), jnp.int32),
            pltpu.VMEM((1024,), jnp.float32),
            pltpu.VMEM((1024,), jnp.float32),
            pltpu.SemaphoreType.DMA,
        ],
        compiler_params=pltpu.CompilerParams(use_tc_tiling_on_sc=False),
    )
    def k(ya_h, yb_h, si_h, di_h, out_h, ia_v, ib_v, va_v, vb_v, sem):
        wid = lax.axis_index("c") * NS + lax.axis_index("s")
        for t in range(PERW // 1024):
            base = wid * PERW + t * 1024
            rowb = wid * (PERW // 128) + t * 8
            pltpu.sync_copy(si_h.at[pl.ds(rowb, 8)], ia_v)
            pltpu.sync_copy(di_h.at[pl.ds(rowb, 8)], ib_v)
            descs = []
            for j in range(8):
                descs.append(
                    pltpu.async_copy(ya_h.at[ia_v.at[j]],
                                     va_v.at[pl.ds(j * 128, 128)], sem))
                descs.append(
                    pltpu.async_copy(yb_h.at[ib_v.at[j]],
                                     vb_v.at[pl.ds(j * 128, 128)], sem))
            for d in descs:
                d.wait()

            def body(i, carry):
                off = pl.multiple_of(i * 16, 16)
                z = va_v[pl.ds(off, 16)] + vb_v[pl.ds(off, 16)]
                va_v[pl.ds(off, 16)] = 1.0 / (1.0 + jnp.exp(-z))
                return carry

            lax.fori_loop(0, 1024 // 16, body, 0)
            pltpu.sync_copy(va_v, out_h.at[pl.ds(base, 1024)])

    return k(ya1, yb1, src2d, dst2d)


def kernel(x, edge_index, edge_attr, w11, b11, w21, b21, root1, bias1,
           w12, b12, w22, b22, root2, bias2, fc_w, fc_b):
    f32 = jnp.float32
    i32 = jnp.int32
    src = edge_index[0].astype(i32)
    dst = edge_index[1].astype(i32)

    w2m1 = w21.reshape(H * H, DIN, H).transpose(1, 2, 0).reshape(DIN, 512)
    b2r1 = b21.reshape(DIN, H)
    w2m2 = w22.reshape(H * H, H, H // 2).transpose(1, 2, 0).reshape(H, 256)
    b2r2 = b22.reshape(H, H // 2)
    s1m = (jnp.arange(512)[:, None] // 64 == jnp.arange(H)[None, :]).astype(f32)
    s2m = (jnp.arange(256)[:, None] // 64
           == jnp.arange(H // 2)[None, :]).astype(f32)
    b11r = b11.reshape(1, H * H)
    b12r = b12.reshape(1, H * H)
    bias1r = bias1.reshape(1, H)
    bias2r = bias2.reshape(1, H // 2)
    fca = fc_w[:H // 2]
    fcb = fc_w[H // 2:]
    fcbias = fc_b.reshape(1, 1)

    pad = EPAD - E
    src2d = jnp.concatenate([src, jnp.zeros((pad,), i32)]).reshape(
        EPAD // 128, 128)
    dst2d = jnp.concatenate([dst, jnp.full((pad,), N, i32)]).reshape(
        EPAD // 128, 128)
    zer = jnp.zeros((NPAD, W16), f32)

    xj = _sc_gather(x, src2d, 512, W128)

    def edge_call(body, xj_arr, win, *weights):
        return pl.pallas_call(
            body,
            grid=(E // EB,),
            in_specs=[
                pl.BlockSpec((EB, DE), lambda i: (i, 0)),
                pl.BlockSpec((EB, win), lambda i: (i, 0)),
            ] + [pl.BlockSpec(w.shape, lambda i: (0,) * w.ndim)
                 for w in weights],
            out_specs=pl.BlockSpec((EB, W16), lambda i: (i, 0)),
            out_shape=jax.ShapeDtypeStruct((EPAD, W16), f32),
        )(edge_attr, xj_arr, *weights)

    m1 = edge_call(_edge1_body, xj, W128, w11, b11r, w2m1, b2r1, s1m)

    s1 = _sc_scatter_sum(m1, dst2d, zer)

    half_a = pl.BlockSpec((None, NB, W16), lambda i: (0, i, 0))
    half_b = pl.BlockSpec((None, NB, W16), lambda i: (1, i, 0))

    x1p, r2 = pl.pallas_call(
        _node1_body,
        grid=(N // NB,),
        in_specs=[
            half_a, half_b,
            pl.BlockSpec((NB, DIN), lambda i: (i, 0)),
            pl.BlockSpec(root1.shape, lambda i: (0, 0)),
            pl.BlockSpec(bias1r.shape, lambda i: (0, 0)),
            pl.BlockSpec(root2.shape, lambda i: (0, 0)),
            pl.BlockSpec(bias2r.shape, lambda i: (0, 0)),
        ],
        out_specs=[
            pl.BlockSpec((NB, W16), lambda i: (i, 0)),
            pl.BlockSpec((NB, H // 2), lambda i: (i, 0)),
        ],
        out_shape=[
            jax.ShapeDtypeStruct((N, W16), f32),
            jax.ShapeDtypeStruct((N, H // 2), f32),
        ],
    )(s1, s1, x, root1, bias1r, root2, bias2r)

    xj1 = _sc_gather(x1p, src2d, 1024, W16)

    m2 = edge_call(_edge2_body, xj1, W16, w12, b12r, w2m2, b2r2, s2m)

    s2 = _sc_scatter_sum(m2, dst2d, zer)

    ya2d, yb2d = pl.pallas_call(
        _node2_body,
        grid=(N // NB,),
        in_specs=[
            half_a, half_b, half_a, half_b,
            pl.BlockSpec((NB, H // 2), lambda i: (i, 0)),
            pl.BlockSpec(fca.shape, lambda i: (0, 0)),
            pl.BlockSpec(fcb.shape, lambda i: (0, 0)),
            pl.BlockSpec(fcbias.shape, lambda i: (0, 0)),
        ],
        out_specs=[
            pl.BlockSpec((NB, 16), lambda i: (i, 0)),
            pl.BlockSpec((NB, 16), lambda i: (i, 0)),
        ],
        out_shape=[
            jax.ShapeDtypeStruct((NPAD, 16), f32),
            jax.ShapeDtypeStruct((NPAD, 16), f32),
        ],
    )(s2, s2, s1, s1, r2, fca, fcb, fcbias)

    out = _sc_final(ya2d[:, 0], yb2d[:, 0], src2d, dst2d)
    return out[:E].reshape(E, 1)
